# baseline (speedup 1.0000x reference)
import functools

import jax
import jax.numpy as jnp
from jax import lax
from jax.experimental import pallas as pl
from jax.experimental.pallas import tpu as pltpu
from jax.experimental.pallas import tpu_sc as plsc

N = 10000
NPAD = 10240
E = 320000
EPAD = 327680
DIM = 128
NLEVELS = 7
NGATES = 3
NGRP = NLEVELS * NGATES
NCAP = 512
ECAP = 12288
EL = NGATES * ECAP
NL = NGATES * NCAP
NW = 32
EW = EL // NW
ECH = 128
NCHUNKS = EW // ECH
NLW = NL // NW
HBINS = NPAD * 8
ZROW = N + 1
ZB = NL

@functools.cache
def _mesh():
    return plsc.VectorSubcoreMesh(core_axis_name="c", subcore_axis_name="s")


def _hist_body(idxc_hbm, idxd_hbm, zeros_hbm, out_c, out_d,
               idx_c, idx_d, ones_v, sem_h, c_sh, d_sh):
    c = lax.axis_index("c")
    s = lax.axis_index("s")
    w = s * 2 + c
    for i in range(8):
        ones_v[pl.ds(i * 16, 16)] = jnp.ones((16,), jnp.float32)

    @pl.when(s == 0)
    def _():
        pltpu.sync_copy(zeros_hbm, c_sh)
        pltpu.sync_copy(zeros_hbm, d_sh)

    plsc.subcore_barrier()
    rows_w = EPAD // 128 // NW

    def chunk(ci, carry):
        base = w * rows_w + ci * 8
        pltpu.sync_copy(idxc_hbm.at[pl.ds(base, 8)], idx_c)
        pltpu.sync_copy(idxd_hbm.at[pl.ds(base, 8)], idx_d)
        cps = []
        for jj in range(8):
            cps.append(pltpu.async_copy(ones_v, c_sh.at[idx_c.at[jj]],
                                        sem_h, add=True))
            cps.append(pltpu.async_copy(ones_v, d_sh.at[idx_d.at[jj]],
                                        sem_h, add=True))
        for cp in cps:
            cp.wait()
        return carry

    lax.fori_loop(0, rows_w // 8, chunk, 0)
    plsc.subcore_barrier()

    @pl.when(s == 0)
    def _():
        pltpu.sync_copy(c_sh, out_c.at[c])
        pltpu.sync_copy(d_sh, out_d.at[c])


@functools.cache
def _hist_call():
    return pl.kernel(
    _hist_body,
    out_type=(jax.ShapeDtypeStruct((2, HBINS), jnp.float32),
              jax.ShapeDtypeStruct((2, HBINS), jnp.float32)),
    mesh=_mesh(),
    scratch_types=[
        pltpu.VMEM((8, 128), jnp.int32),
        pltpu.VMEM((8, 128), jnp.int32),
        pltpu.VMEM((128,), jnp.float32),
        pltpu.SemaphoreType.DMA,
        pltpu.VMEM_SHARED((HBINS,), jnp.float32),
        pltpu.VMEM_SHARED((HBINS,), jnp.float32),
    ],
    )


def _step_body(hs_hbm, hnew_hbm, idxe_hbm, idxa_hbm, idxb_hbm, sidx_hbm,
               hf_hbm, xs_hs_hbm, xs_hfa_hbm, xs_hfb_hbm,
               idx_e, idx_a, idx_b, idx_n, rows_n,
               hs0, hs1, fa0, fa1, fb0, fb1,
               sem_s, sem_g0, sem_g1, sem_w0, sem_w1):
    c = lax.axis_index("c")
    s = lax.axis_index("s")
    w = s * 2 + c

    nb = w * NLW
    pltpu.sync_copy(sidx_hbm.at[pl.ds(nb, NLW)], idx_n)
    pltpu.sync_copy(hnew_hbm.at[pl.ds(nb, NLW)], rows_n)
    cp_sc = pltpu.async_copy(rows_n, hf_hbm.at[idx_n], sem_s)

    pltpu.sync_copy(idxe_hbm.at[w], idx_e)
    pltpu.sync_copy(idxa_hbm.at[w], idx_a)
    pltpu.sync_copy(idxb_hbm.at[w], idx_b)

    hsb = (hs0, hs1)
    fab = (fa0, fa1)
    fbb = (fb0, fb1)
    sems = (sem_g0, sem_g1)
    wsems = (sem_w0, sem_w1)
    fired = []
    for ci in range(NCHUNKS):
        p = ci % 2
        if ci >= 2:
            for cp in fired[ci - 2][1]:
                cp.wait()
        g = (pltpu.async_copy(hs_hbm.at[idx_e.at[ci]], hsb[p], sems[p]),
             pltpu.async_copy(hf_hbm.at[idx_a.at[ci]], fab[p], sems[p]),
             pltpu.async_copy(hnew_hbm.at[idx_b.at[ci]], fbb[p], sems[p]))
        if ci >= 1:
            for cp in fired[ci - 1][0]:
                cp.wait()
            base = w * EW + (ci - 1) * ECH
            q = (ci - 1) % 2
            ws = (pltpu.async_copy(hsb[q], xs_hs_hbm.at[pl.ds(base, ECH)],
                                   wsems[q]),
                  pltpu.async_copy(fab[q], xs_hfa_hbm.at[pl.ds(base, ECH)],
                                   wsems[q]),
                  pltpu.async_copy(fbb[q], xs_hfb_hbm.at[pl.ds(base, ECH)],
                                   wsems[q]))
            fired[ci - 1] = (fired[ci - 1][0], ws)
        fired.append((g, ()))
    last = NCHUNKS - 1
    for cp in fired[last][0]:
        cp.wait()
    base = w * EW + last * ECH
    q = last % 2
    ws = (pltpu.async_copy(hsb[q], xs_hs_hbm.at[pl.ds(base, ECH)], wsems[q]),
          pltpu.async_copy(fab[q], xs_hfa_hbm.at[pl.ds(base, ECH)], wsems[q]),
          pltpu.async_copy(fbb[q], xs_hfb_hbm.at[pl.ds(base, ECH)], wsems[q]))
    for cp in fired[last - 1][1]:
        cp.wait()
    for cp in ws:
        cp.wait()
    cp_sc.wait()


@functools.cache
def _step_call():
    return pl.kernel(
    _step_body,
    out_type=(jax.ShapeDtypeStruct((EL, DIM), jnp.float32),
              jax.ShapeDtypeStruct((EL, DIM), jnp.float32),
              jax.ShapeDtypeStruct((EL, DIM), jnp.float32)),
    mesh=_mesh(),
    scratch_types=[
        pltpu.VMEM((NCHUNKS, ECH), jnp.int32),
        pltpu.VMEM((NCHUNKS, ECH), jnp.int32),
        pltpu.VMEM((NCHUNKS, ECH), jnp.int32),
        pltpu.VMEM((NLW,), jnp.int32),
        pltpu.VMEM((NLW, DIM), jnp.float32),
        pltpu.VMEM((ECH, DIM), jnp.float32),
        pltpu.VMEM((ECH, DIM), jnp.float32),
        pltpu.VMEM((ECH, DIM), jnp.float32),
        pltpu.VMEM((ECH, DIM), jnp.float32),
        pltpu.VMEM((ECH, DIM), jnp.float32),
        pltpu.VMEM((ECH, DIM), jnp.float32),
        pltpu.SemaphoreType.DMA,
        pltpu.SemaphoreType.DMA,
        pltpu.SemaphoreType.DMA,
        pltpu.SemaphoreType.DMA,
        pltpu.SemaphoreType.DMA,
    ],
    )


def _scatter_body(hnew_hbm, nidx_hbm, hf_hbm, idx_n, rows_n, sem):
    c = lax.axis_index("c")
    s = lax.axis_index("s")
    w = s * 2 + c
    nb = w * NLW
    pltpu.sync_copy(nidx_hbm.at[pl.ds(nb, NLW)], idx_n)
    pltpu.sync_copy(hnew_hbm.at[pl.ds(nb, NLW)], rows_n)
    pltpu.async_copy(rows_n, hf_hbm.at[idx_n], sem).wait()


@functools.cache
def _scatter_call():
    return pl.kernel(
    _scatter_body,
    out_type=(),
    mesh=_mesh(),
    scratch_types=[
        pltpu.VMEM((NLW,), jnp.int32),
        pltpu.VMEM((NLW, DIM), jnp.float32),
        pltpu.SemaphoreType.DMA,
    ],
    )


def _enc_body(c_ref, d_ref, ws1_ref, ws2_ref, wt1_ref, wt2_ref,
              whss_ref, whst_ref, bhs_ref, out_ref):
    cm = c_ref[0] + c_ref[1]
    dm = d_ref[0] + d_ref[1]
    a1 = jax.nn.relu(ws1_ref[...])
    a2 = jax.nn.relu(wt1_ref[...])
    s = jax.nn.relu(jnp.dot(jnp.dot(cm, a1), ws2_ref[...]))
    t = jax.nn.relu(jnp.dot(jnp.dot(dm, a2), wt2_ref[...]))
    out_ref[...] = (jnp.dot(s, whss_ref[...]) + jnp.dot(t, whst_ref[...])
                    + bhs_ref[...])


def _enc_call(c2, d2, ws1p, ws2, wt1p, wt2, whss, whst, bhs):
    blk = 1280
    grid = NPAD // blk
    return pl.pallas_call(
        _enc_body,
        grid=(grid,),
        in_specs=[
            pl.BlockSpec((2, blk, 8), lambda b: (0, b, 0)),
            pl.BlockSpec((2, blk, 8), lambda b: (0, b, 0)),
            pl.BlockSpec((8, DIM), lambda b: (0, 0)),
            pl.BlockSpec((DIM, DIM), lambda b: (0, 0)),
            pl.BlockSpec((8, DIM), lambda b: (0, 0)),
            pl.BlockSpec((DIM, DIM), lambda b: (0, 0)),
            pl.BlockSpec((DIM, DIM), lambda b: (0, 0)),
            pl.BlockSpec((DIM, DIM), lambda b: (0, 0)),
            pl.BlockSpec((1, DIM), lambda b: (0, 0)),
        ],
        out_specs=pl.BlockSpec((blk, DIM), lambda b: (b, 0)),
        out_shape=jax.ShapeDtypeStruct((NPAD, DIM), jnp.float32),
    )(c2, d2, ws1p, ws2, wt1p, wt2, whss, whst, bhs)


_EBLK = 256
_NMSG = EL // _EBLK


def _msg_gru_body(xs_hs_ref, xs_hfa_ref, xs_hfb_ref, dslot_ref, wa1s_ref,
                  wa1f_ref, wa2_ref, ba1_ref, ba2_ref, wih_ref,
                  bih_ref, bhh_ref, out_ref, acc_ref):
    b = pl.program_id(0)

    @pl.when(b == 0)
    def _():
        acc_ref[...] = jnp.zeros_like(acc_ref)

    @pl.when(b < _NMSG)
    def _():
        g = b // (_NMSG // NGATES)
        x2 = xs_hfa_ref[...] + xs_hfb_ref[...]
        h1 = jax.nn.relu(jnp.dot(xs_hs_ref[...], wa1s_ref[0])
                         + jnp.dot(x2, wa1f_ref[0]) + ba1_ref[0])
        m = jnp.dot(h1, wa2_ref[0]) + ba2_ref[0]
        slots = dslot_ref[0, 0, :]
        iot = lax.broadcasted_iota(jnp.int32, (NCAP, _EBLK), 0)
        pt = (iot == slots[None, :]).astype(jnp.float32)
        acc_ref[pl.ds(g, 1)] += jnp.dot(pt, m)[None]

    @pl.when(b >= _NMSG)
    def _():
        g = b - _NMSG
        msg = acc_ref[pl.ds(g, 1)][0]
        gi = jnp.dot(msg, wih_ref[0]) + bih_ref[0]
        gh = bhh_ref[0]
        r = jax.nn.sigmoid(gi[:, :DIM] + gh[:, :DIM])
        z = jax.nn.sigmoid(gi[:, DIM:2 * DIM] + gh[:, DIM:2 * DIM])
        nn = jnp.tanh(gi[:, 2 * DIM:] + r * gh[:, 2 * DIM:])
        out_ref[0] = (1.0 - z) * nn


def _msg_gru_call(xs_hs, xs_hfa, xs_hfb, dslot, wa1s, wa1f, wa2, ba1, ba2,
                  wih, bih, bhh):
    nblk = _NMSG // NGATES
    mcap = _NMSG - 1

    def _gmsg(b):
        return jnp.minimum(b // nblk, NGATES - 1)

    def _ggru(b):
        return jnp.maximum(b - _NMSG, 0)

    return pl.pallas_call(
        _msg_gru_body,
        grid=(_NMSG + NGATES,),
        in_specs=[
            pl.BlockSpec((_EBLK, DIM), lambda b: (jnp.minimum(b, mcap), 0)),
            pl.BlockSpec((_EBLK, DIM), lambda b: (jnp.minimum(b, mcap), 0)),
            pl.BlockSpec((_EBLK, DIM), lambda b: (jnp.minimum(b, mcap), 0)),
            pl.BlockSpec((1, 1, _EBLK), lambda b: (jnp.minimum(b, mcap), 0, 0)),
            pl.BlockSpec((1, DIM, DIM), lambda b: (_gmsg(b), 0, 0)),
            pl.BlockSpec((1, DIM, DIM), lambda b: (_gmsg(b), 0, 0)),
            pl.BlockSpec((1, DIM, DIM), lambda b: (_gmsg(b), 0, 0)),
            pl.BlockSpec((1, 1, DIM), lambda b: (_gmsg(b), 0, 0)),
            pl.BlockSpec((1, 1, DIM), lambda b: (_gmsg(b), 0, 0)),
            pl.BlockSpec((1, DIM, 3 * DIM), lambda b: (_ggru(b), 0, 0)),
            pl.BlockSpec((1, 1, 3 * DIM), lambda b: (_ggru(b), 0, 0)),
            pl.BlockSpec((1, 1, 3 * DIM), lambda b: (_ggru(b), 0, 0)),
        ],
        out_specs=pl.BlockSpec((1, NCAP, DIM), lambda b: (_ggru(b), 0, 0)),
        out_shape=jax.ShapeDtypeStruct((NGATES, NCAP, DIM), jnp.float32),
        scratch_shapes=[pltpu.VMEM((NGATES, NCAP, DIM), jnp.float32)],
    )(xs_hs, xs_hfa, xs_hfb, dslot, wa1s, wa1f, wa2, ba1, ba2, wih,
      bih, bhh)


def _plan(gate, lvl, src, dst):
    gmap = jnp.array([21, 21, 1, 0, 21, 2], jnp.int32)[gate]
    gid = jnp.where((lvl >= 1) & (gmap < NGATES),
                    (lvl - 1) * NGATES + gmap, NGRP).astype(jnp.int32)

    order = jnp.argsort(gid, stable=True).astype(jnp.int32)
    gs = gid[order]
    cn = jnp.bincount(gid, length=NGRP + 1)
    st = jnp.concatenate([jnp.zeros((1,), cn.dtype), jnp.cumsum(cn)])
    pos = jnp.arange(N, dtype=jnp.int32) - st[gs].astype(jnp.int32)
    vn = (gs < NGRP) & (pos < NCAP)
    tgt = jnp.where(vn, gs * NCAP + pos, NGRP * NCAP)
    nidx = jnp.full((NGRP * NCAP + 1,), N, jnp.int32).at[tgt].set(
        order)[:NGRP * NCAP]
    slot = jnp.zeros((N,), jnp.int32).at[order].set(
        jnp.where(vn, pos, NCAP))

    egid = gid[dst]
    eord = jnp.argsort(egid, stable=True).astype(jnp.int32)
    egs = egid[eord]
    ce = jnp.bincount(egid, length=NGRP + 1)
    ste = jnp.concatenate([jnp.zeros((1,), ce.dtype), jnp.cumsum(ce)])
    epos = jnp.arange(E, dtype=jnp.int32) - ste[egs].astype(jnp.int32)
    ve = (egs < NGRP) & (epos < ECAP)
    etgt = jnp.where(ve, egs * ECAP + epos, NGRP * ECAP)
    esrc = jnp.full((NGRP * ECAP + 1,), ZROW, jnp.int32).at[etgt].set(
        src[eord])[:NGRP * ECAP]
    edsl = jnp.full((NGRP * ECAP + 1,), NCAP, jnp.int32).at[etgt].set(
        slot[dst[eord]])[:NGRP * ECAP]

    esrc7 = esrc.reshape(NLEVELS, EL)
    updlvl = jnp.where(gid < NGRP, gid // NGATES, 99).astype(jnp.int32)
    posb = jnp.where(slot < NCAP, (gid % NGATES) * NCAP + slot, ZB)
    updlvl_p = jnp.concatenate(
        [updlvl, jnp.full((NPAD + 1 - N,), 99, jnp.int32)])
    posb_p = jnp.concatenate(
        [posb.astype(jnp.int32), jnp.full((NPAD + 1 - N,), ZB, jnp.int32)])
    lv = jnp.arange(NLEVELS, dtype=jnp.int32)[:, None]
    ua = updlvl_p[esrc7]
    hit = ua == (lv - 1)
    idxa = jnp.where(hit, ZROW, esrc7)
    idxb = jnp.where(hit, posb_p[esrc7], ZB).astype(jnp.int32)

    return (nidx.reshape(NLEVELS, NL),
            esrc7.reshape(NLEVELS, NW, NCHUNKS, ECH),
            idxa.reshape(NLEVELS, NW, NCHUNKS, ECH),
            idxb.reshape(NLEVELS, NW, NCHUNKS, ECH),
            edsl.reshape(NLEVELS, _NMSG, 1, _EBLK))


def kernel(xag_x, xag_edge_index, xag_gate, xag_forward_level,
           xag_forward_index, params):
    del xag_forward_index
    src = xag_edge_index[0]
    dst = xag_edge_index[1]
    gate = xag_gate[:, 0]
    x1 = xag_x[:, 1]

    nidx_all, esrc_all, idxa_all, idxb_all, edsl_all = _plan(
        gate, xag_forward_level, src, dst)

    pad_e = jnp.full((EPAD - E,), N * 8, jnp.int32)
    idxc = jnp.concatenate([dst * 8 + x1[src], pad_e]).reshape(-1, 128)
    idxd = jnp.concatenate([src * 8 + x1[dst], pad_e]).reshape(-1, 128)
    zeros_h = jnp.zeros((HBINS,), jnp.float32)
    c2, d2 = _hist_call()(idxc, idxd, zeros_h)

    pad_w = jnp.zeros((2, DIM), jnp.float32)
    ws1p = jnp.concatenate([params['W_s1'], pad_w])
    wt1p = jnp.concatenate([params['W_t1'], pad_w])
    hs_pad = _enc_call(
        c2.reshape(2, NPAD, 8), d2.reshape(2, NPAD, 8),
        ws1p, params['W_s2'], wt1p, params['W_t2'],
        params['W_hs'][:DIM], params['W_hs'][DIM:],
        params['b_hs'][None])

    names = ('and', 'not', 'xor')
    wa1 = jnp.stack([params[n]['Wa1'] for n in names])
    wa1s, wa1f = wa1[:, :DIM], wa1[:, DIM:]
    wa2 = jnp.stack([params[n]['Wa2'] for n in names])
    ba1 = jnp.stack([params[n]['ba1'] for n in names])[:, None]
    ba2 = jnp.stack([params[n]['ba2'] for n in names])[:, None]
    wih = jnp.stack([params[n]['Wih'] for n in names])
    bih = jnp.stack([params[n]['bih'] for n in names])[:, None]
    bhh = jnp.stack([params[n]['bhh'] for n in names])[:, None]

    hf_ref = jax.new_ref(jnp.zeros((NPAD, DIM), jnp.float32))
    zpad = jnp.zeros((ZB + 8 - NL, DIM), jnp.float32)
    hnew_pad = jnp.zeros((ZB + 8, DIM), jnp.float32)
    sidx = jnp.full((NL,), N, jnp.int32)
    for l in range(NLEVELS):
        xs_hs, xs_hfa, xs_hfb = _step_call()(
            hs_pad, hnew_pad, esrc_all[l], idxa_all[l], idxb_all[l], sidx,
            hf_ref)
        hnew = _msg_gru_call(xs_hs, xs_hfa, xs_hfb, edsl_all[l], wa1s,
                             wa1f, wa2, ba1, ba2, wih, bih, bhh)
        hnew_pad = jnp.concatenate([hnew.reshape(NL, DIM), zpad])
        sidx = nidx_all[l]
    _scatter_call()(hnew_pad[:NL], sidx, hf_ref)

    hf = hf_ref[...]
    return hs_pad[:N], hf[:N]
